# Initial kernel scaffold; baseline (speedup 1.0000x reference)
import jax
import jax.numpy as jnp
from jax.experimental import pallas as pl


def kernel(x, batch_index):
    raise NotImplementedError("write your pallas kernel here")


# speedup vs baseline: 5.6513x; 5.6513x over previous
import functools

import jax
import jax.numpy as jnp
from jax import lax
from jax.experimental import pallas as pl
from jax.experimental.pallas import tpu as pltpu
from jax.experimental.pallas import tpu_sc as plsc

N_ROWS = 320000
D = 128
S = 512
NC, NS = 2, 16
NW = NC * NS
ROWS_PER_TILE = N_ROWS // NW
C = 80
NCHUNK = ROWS_PER_TILE // C
SEG_PER_TILE = S // NS
CW = 16
LANES = 16


def _sc_partial_segsum(x, idx2d, ones_cw):
  mesh = plsc.VectorSubcoreMesh(
      core_axis_name="c", subcore_axis_name="s", num_cores=NC, num_subcores=NS)

  @functools.partial(
      pl.kernel,
      out_type=(
          jax.ShapeDtypeStruct((NC * S, D), jnp.float32),
          jax.ShapeDtypeStruct((NC * S, CW), jnp.float32),
      ),
      mesh=mesh,
      compiler_params=pltpu.CompilerParams(use_tc_tiling_on_sc=False),
      scratch_types=[
          pltpu.VMEM((NCHUNK, C), jnp.int32),
          pltpu.VMEM((C, D), jnp.float32),
          pltpu.VMEM((C, CW), jnp.float32),
          pltpu.VMEM((SEG_PER_TILE, CW), jnp.float32),
          pltpu.VMEM_SHARED((S, D), jnp.float32),
          pltpu.VMEM_SHARED((S, CW), jnp.float32),
      ],
  )
  def k(x_hbm, idx_hbm, ones_hbm, psums_hbm, pcnts_hbm, idx_v, xbuf, ones_v,
        zc, sums_sh, cnts_sh):
    cid = lax.axis_index("c")
    sid = lax.axis_index("s")
    wid = cid * NS + sid
    row0 = wid * ROWS_PER_TILE

    pltpu.sync_copy(idx_hbm.at[wid], idx_v)

    zeros16 = jnp.zeros((LANES,), jnp.float32)

    def zrow(i, _):
      xbuf[i // (D // LANES), pl.ds((i % (D // LANES)) * LANES, LANES)] = zeros16
      return 0
    lax.fori_loop(0, SEG_PER_TILE * (D // LANES), zrow, 0)
    pltpu.sync_copy(ones_hbm.at[pl.ds(0, C)], ones_v)
    pltpu.sync_copy(ones_hbm.at[pl.ds(C, SEG_PER_TILE)], zc)

    pltpu.sync_copy(xbuf.at[pl.ds(0, SEG_PER_TILE)],
                    sums_sh.at[pl.ds(sid * SEG_PER_TILE, SEG_PER_TILE)])
    pltpu.sync_copy(zc, cnts_sh.at[pl.ds(sid * SEG_PER_TILE, SEG_PER_TILE)])
    plsc.subcore_barrier()

    def body(j, _):
      pltpu.sync_copy(x_hbm.at[pl.ds(row0 + j * C, C)], xbuf)
      pltpu.sync_copy(xbuf, sums_sh.at[idx_v.at[j]], add=True)
      pltpu.sync_copy(ones_v, cnts_sh.at[idx_v.at[j]], add=True)
      return 0
    lax.fori_loop(0, NCHUNK, body, 0)
    plsc.subcore_barrier()

    pltpu.sync_copy(sums_sh.at[pl.ds(sid * SEG_PER_TILE, SEG_PER_TILE)],
                    xbuf.at[pl.ds(0, SEG_PER_TILE)])
    pltpu.sync_copy(xbuf.at[pl.ds(0, SEG_PER_TILE)],
                    psums_hbm.at[pl.ds(cid * S + sid * SEG_PER_TILE,
                                       SEG_PER_TILE)])
    pltpu.sync_copy(cnts_sh.at[pl.ds(sid * SEG_PER_TILE, SEG_PER_TILE)], zc)
    pltpu.sync_copy(zc, pcnts_hbm.at[pl.ds(cid * S + sid * SEG_PER_TILE,
                                           SEG_PER_TILE)])

  return k(x, idx2d, ones_cw)


def _combine(psums, pcnts):
  def body(ps_ref, pc_ref, o_ref):
    sums = ps_ref[0] + ps_ref[1]
    cnts = pc_ref[0, :, 0:1] + pc_ref[1, :, 0:1]
    o_ref[...] = sums / cnts
  return pl.pallas_call(
      body,
      out_shape=jax.ShapeDtypeStruct((S, D), jnp.float32),
  )(psums.reshape(NC, S, D), pcnts.reshape(NC, S, CW))


def kernel(x, batch_index):
  idx2d = batch_index.astype(jnp.int32).reshape(NW, NCHUNK, C)
  ones_cw = jnp.concatenate([jnp.ones((C, CW), jnp.float32),
                             jnp.zeros((SEG_PER_TILE, CW), jnp.float32)])
  psums, pcnts = _sc_partial_segsum(x, idx2d, ones_cw)
  return _combine(psums, pcnts)


# speedup vs baseline: 7.1938x; 1.2729x over previous
import functools

import jax
import jax.numpy as jnp
from jax import lax
from jax.experimental import pallas as pl
from jax.experimental.pallas import tpu as pltpu
from jax.experimental.pallas import tpu_sc as plsc

N_ROWS = 320000
D = 128
S = 512
NC, NS = 2, 16
NW = NC * NS
ROWS_PER_TILE = N_ROWS // NW
C = 125
NCHUNK = ROWS_PER_TILE // C
NPAIR = NCHUNK // 2
SEG_PER_TILE = S // NS
CW = 16
LANES = 16


def _sc_partial_segsum(x, idx2d, ones_cw):
  mesh = plsc.VectorSubcoreMesh(
      core_axis_name="c", subcore_axis_name="s", num_cores=NC, num_subcores=NS)

  @functools.partial(
      pl.kernel,
      out_type=(
          jax.ShapeDtypeStruct((NC * S, D), jnp.float32),
          jax.ShapeDtypeStruct((NC * S, CW), jnp.float32),
      ),
      mesh=mesh,
      compiler_params=pltpu.CompilerParams(use_tc_tiling_on_sc=False),
      scratch_types=[
          pltpu.VMEM((NCHUNK, C), jnp.int32),
          pltpu.VMEM((C, D), jnp.float32),
          pltpu.VMEM((C, D), jnp.float32),
          pltpu.VMEM((C, CW), jnp.float32),
          pltpu.VMEM((SEG_PER_TILE, CW), jnp.float32),
          pltpu.VMEM_SHARED((S, D), jnp.float32),
          pltpu.VMEM_SHARED((S, CW), jnp.float32),
          pltpu.SemaphoreType.DMA,
          pltpu.SemaphoreType.DMA,
      ],
  )
  def k(x_hbm, idx_hbm, ones_hbm, psums_hbm, pcnts_hbm, idx_v, xbuf0, xbuf1,
        ones_v, zc, sums_sh, cnts_sh, gsem0, gsem1):
    cid = lax.axis_index("c")
    sid = lax.axis_index("s")
    wid = cid * NS + sid
    row0 = wid * ROWS_PER_TILE

    pltpu.sync_copy(idx_hbm.at[wid], idx_v)

    zeros16 = jnp.zeros((LANES,), jnp.float32)

    def zrow(i, _):
      xbuf0[i // (D // LANES), pl.ds((i % (D // LANES)) * LANES, LANES)] = zeros16
      return 0
    lax.fori_loop(0, SEG_PER_TILE * (D // LANES), zrow, 0)
    pltpu.sync_copy(ones_hbm.at[pl.ds(0, C)], ones_v)
    pltpu.sync_copy(ones_hbm.at[pl.ds(C, SEG_PER_TILE)], zc)

    pltpu.sync_copy(xbuf0.at[pl.ds(0, SEG_PER_TILE)],
                    sums_sh.at[pl.ds(sid * SEG_PER_TILE, SEG_PER_TILE)])
    pltpu.sync_copy(zc, cnts_sh.at[pl.ds(sid * SEG_PER_TILE, SEG_PER_TILE)])
    plsc.subcore_barrier()

    def gstart(j, buf, sem):
      pltpu.async_copy(x_hbm.at[pl.ds(row0 + j * C, C)], buf, sem)

    def gwait(buf, sem):
      pltpu.make_async_copy(x_hbm.at[pl.ds(0, C)], buf, sem).wait()

    def scat(j, buf):
      pltpu.sync_copy(buf, sums_sh.at[idx_v.at[j]], add=True)
      pltpu.sync_copy(ones_v, cnts_sh.at[idx_v.at[j]], add=True)

    gstart(0, xbuf0, gsem0)
    def pair(p, _):
      j0 = 2 * p
      gwait(xbuf0, gsem0)
      gstart(j0 + 1, xbuf1, gsem1)
      scat(j0, xbuf0)
      gwait(xbuf1, gsem1)
      @pl.when(p < NPAIR - 1)
      def _():
        gstart(j0 + 2, xbuf0, gsem0)
      scat(j0 + 1, xbuf1)
      return 0
    lax.fori_loop(0, NPAIR, pair, 0)
    plsc.subcore_barrier()

    pltpu.sync_copy(sums_sh.at[pl.ds(sid * SEG_PER_TILE, SEG_PER_TILE)],
                    xbuf0.at[pl.ds(0, SEG_PER_TILE)])
    pltpu.sync_copy(xbuf0.at[pl.ds(0, SEG_PER_TILE)],
                    psums_hbm.at[pl.ds(cid * S + sid * SEG_PER_TILE,
                                       SEG_PER_TILE)])
    pltpu.sync_copy(cnts_sh.at[pl.ds(sid * SEG_PER_TILE, SEG_PER_TILE)], zc)
    pltpu.sync_copy(zc, pcnts_hbm.at[pl.ds(cid * S + sid * SEG_PER_TILE,
                                           SEG_PER_TILE)])

  return k(x, idx2d, ones_cw)


def _combine(psums, pcnts):
  def body(ps_ref, pc_ref, o_ref):
    sums = ps_ref[0] + ps_ref[1]
    cnts = pc_ref[0, :, 0:1] + pc_ref[1, :, 0:1]
    o_ref[...] = sums / cnts
  return pl.pallas_call(
      body,
      out_shape=jax.ShapeDtypeStruct((S, D), jnp.float32),
  )(psums.reshape(NC, S, D), pcnts.reshape(NC, S, CW))


def kernel(x, batch_index):
  idx2d = batch_index.astype(jnp.int32).reshape(NW, NCHUNK, C)
  ones_cw = jnp.concatenate([jnp.ones((C, CW), jnp.float32),
                             jnp.zeros((SEG_PER_TILE, CW), jnp.float32)])
  psums, pcnts = _sc_partial_segsum(x, idx2d, ones_cw)
  return _combine(psums, pcnts)
